# baseline (speedup 1.0000x reference)
import jax
import jax.numpy as jnp
from jax.experimental import pallas as pl
from jax.experimental.pallas import tpu as pltpu

_K = 64


def _bcast_body(table_ref, out_ref, buf, sem):
    buf[...] = jnp.broadcast_to(table_ref[...][None, :, :], buf.shape)
    batch = out_ref.shape[0]
    n = batch // _K
    copies = [
        pltpu.make_async_copy(buf, out_ref.at[pl.ds(j * _K, _K)], sem)
        for j in range(n)
    ]
    for c in copies:
        c.start()
    for c in copies:
        c.wait()


def kernel(x, table):
    batch, seq_len = x.shape
    embed_dim = table.shape[1]
    table_slice = jax.lax.slice(table, (0, 0), (seq_len, embed_dim))
    return pl.pallas_call(
        _bcast_body,
        in_specs=[
            pl.BlockSpec((seq_len, embed_dim), lambda: (0, 0)),
        ],
        out_specs=pl.BlockSpec(memory_space=pl.ANY),
        out_shape=jax.ShapeDtypeStruct((batch, seq_len, embed_dim), table.dtype),
        scratch_shapes=[
            pltpu.VMEM((_K, seq_len, embed_dim), table.dtype),
            pltpu.SemaphoreType.DMA,
        ],
    )(table_slice)
